# baseline (speedup 1.0000x reference)
import numpy as np

import jax
import jax.numpy as jnp
from jax.experimental import pallas as pl
from jax.sharding import Mesh, PartitionSpec as P


def _add_body(x_ref, t_ref, o_ref):
    o_ref[...] = x_ref[...] + t_ref[...][None, :, :]


def _tc_add(inputs, pos_table):
    B, S, D = inputs.shape
    BS = 512
    return pl.pallas_call(
        _add_body,
        grid=(S // BS,),
        in_specs=[
            pl.BlockSpec((B, BS, D), lambda i: (0, i, 0)),
            pl.BlockSpec((BS, D), lambda i: (i, 0)),
        ],
        out_specs=pl.BlockSpec((B, BS, D), lambda i: (0, i, 0)),
        out_shape=jax.ShapeDtypeStruct((B, S, D), inputs.dtype),
    )(inputs, pos_table)


def kernel(inputs, pos_table):
    devs = jax.devices()
    n = 2 if len(devs) >= 2 else 1
    if n == 1:
        return _tc_add(inputs, pos_table)
    mesh = Mesh(np.asarray(devs[:n]), ("s",))
    f = jax.shard_map(
        _tc_add,
        mesh=mesh,
        in_specs=(P(None, "s", None), P("s", None)),
        out_specs=P(None, "s", None),
        check_vma=False,
    )
    return f(inputs, pos_table)


# speedup vs baseline: 6.1118x; 6.1118x over previous
import jax
import jax.numpy as jnp
from jax.experimental import pallas as pl


def _add_body(x_ref, t_ref, o_ref):
    o_ref[...] = x_ref[...] + t_ref[...]


def kernel(inputs, pos_table):
    B, S, D = inputs.shape
    BS = 512
    n_seq = S // BS
    x2d = inputs.reshape(B * S, D)
    out = pl.pallas_call(
        _add_body,
        grid=(n_seq, B),
        in_specs=[
            pl.BlockSpec((BS, D), lambda j, b: (b * n_seq + j, 0)),
            pl.BlockSpec((BS, D), lambda j, b: (j, 0)),
        ],
        out_specs=pl.BlockSpec((BS, D), lambda j, b: (b * n_seq + j, 0)),
        out_shape=jax.ShapeDtypeStruct((B * S, D), inputs.dtype),
    )(x2d, pos_table)
    return out.reshape(B, S, D)
